# baseline (speedup 1.0000x reference)
import functools

import jax
import jax.numpy as jnp
from jax import lax
from jax.experimental import pallas as pl
from jax.experimental.pallas import tpu as pltpu
from jax.experimental.pallas import tpu_sc as plsc

SEQ = 200
BATCH = 4096
HIDDEN = 64
TOT = SEQ * BATCH
CHUNK = 128
NCHUNKS = TOT // CHUNK
NC = 2
NS = 16
NW = NC * NS
CPW = NCHUNKS // NW
NBUF = 4
LOOK = 2


def _emb_body(idx_hbm, table_hbm, out_hbm, idx_v,
              b0, b1, b2, b3, s0, s1, s2, s3, st0, st1, st2, st3):
    bufs = (b0, b1, b2, b3)
    sems = (s0, s1, s2, s3)
    stsems = (st0, st1, st2, st3)
    wid = lax.axis_index("s") * NC + lax.axis_index("c")
    row0 = wid * CPW

    def out_at(c):
        return out_hbm.at[pl.ds((row0 + c) * CHUNK, CHUNK)]

    def gather(c, b):
        pltpu.make_async_copy(table_hbm.at[idx_v.at[c]], bufs[b], sems[b]).start()

    pltpu.sync_copy(idx_hbm.at[pl.ds(row0, CPW)], idx_v)

    for c in range(LOOK):
        gather(c, c % NBUF)

    def group(g, carry):
        for b in range(NBUF):
            c = g * NBUF + b
            pb = (b + LOOK) % NBUF

            @pl.when(c + LOOK < CPW)
            def _():
                @pl.when(c >= NBUF - LOOK)
                def _():
                    pltpu.make_async_copy(
                        bufs[pb], out_at(c + LOOK - NBUF), stsems[pb]).wait()

                gather(c + LOOK, pb)

            pltpu.make_async_copy(table_hbm.at[idx_v.at[c]], bufs[b], sems[b]).wait()
            pltpu.make_async_copy(bufs[b], out_at(c), stsems[b]).start()
        return carry

    lax.fori_loop(0, CPW // NBUF, group, 0)

    for c in range(CPW - NBUF, CPW):
        b = c % NBUF
        pltpu.make_async_copy(bufs[b], out_at(c), stsems[b]).wait()


def kernel(input_ids, table):
    idx = input_ids.reshape(NCHUNKS, CHUNK).astype(jnp.int32)
    mesh = plsc.VectorSubcoreMesh(core_axis_name="c", subcore_axis_name="s")
    run = functools.partial(
        pl.kernel,
        mesh=mesh,
        compiler_params=pltpu.CompilerParams(use_tc_tiling_on_sc=False),
        out_type=jax.ShapeDtypeStruct((TOT, HIDDEN), jnp.float32),
        scratch_types=[
            pltpu.VMEM((CPW, CHUNK), jnp.int32),
        ] + [pltpu.VMEM((CHUNK, HIDDEN), jnp.float32) for _ in range(NBUF)]
          + [pltpu.SemaphoreType.DMA for _ in range(2 * NBUF)],
    )(_emb_body)
    out = run(idx, table)
    return out.reshape(SEQ, BATCH, HIDDEN)


# speedup vs baseline: 1.0015x; 1.0015x over previous
import functools

import jax
import jax.numpy as jnp
from jax import lax
from jax.experimental import pallas as pl
from jax.experimental.pallas import tpu as pltpu
from jax.experimental.pallas import tpu_sc as plsc

SEQ = 200
BATCH = 4096
HIDDEN = 64
TOT = SEQ * BATCH
CHUNK = 128
NCHUNKS = TOT // CHUNK
NC = 2
NS = 16
NW = NC * NS
CPW = NCHUNKS // NW
NBUF = 8
LOOK = 4


def _emb_body(idx_hbm, table_hbm, out_hbm, idx_v, *rest):
    bufs = rest[:NBUF]
    sems = rest[NBUF:2 * NBUF]
    stsems = rest[2 * NBUF:]
    wid = lax.axis_index("s") * NC + lax.axis_index("c")
    row0 = wid * CPW

    def out_at(c):
        return out_hbm.at[pl.ds((row0 + c) * CHUNK, CHUNK)]

    def gather(c, b):
        pltpu.make_async_copy(table_hbm.at[idx_v.at[c]], bufs[b], sems[b]).start()

    pltpu.sync_copy(idx_hbm.at[pl.ds(row0, CPW)], idx_v)

    for c in range(LOOK):
        gather(c, c % NBUF)

    def group(g, carry):
        for b in range(NBUF):
            c = g * NBUF + b
            pb = (b + LOOK) % NBUF

            @pl.when(c + LOOK < CPW)
            def _():
                @pl.when(c >= NBUF - LOOK)
                def _():
                    pltpu.make_async_copy(
                        bufs[pb], out_at(c + LOOK - NBUF), stsems[pb]).wait()

                gather(c + LOOK, pb)

            pltpu.make_async_copy(table_hbm.at[idx_v.at[c]], bufs[b], sems[b]).wait()
            pltpu.make_async_copy(bufs[b], out_at(c), stsems[b]).start()
        return carry

    lax.fori_loop(0, CPW // NBUF, group, 0)

    for c in range(CPW - NBUF, CPW):
        b = c % NBUF
        pltpu.make_async_copy(bufs[b], out_at(c), stsems[b]).wait()


def kernel(input_ids, table):
    idx = input_ids.reshape(NCHUNKS, CHUNK).astype(jnp.int32)
    mesh = plsc.VectorSubcoreMesh(core_axis_name="c", subcore_axis_name="s")
    run = functools.partial(
        pl.kernel,
        mesh=mesh,
        compiler_params=pltpu.CompilerParams(use_tc_tiling_on_sc=False),
        out_type=jax.ShapeDtypeStruct((TOT, HIDDEN), jnp.float32),
        scratch_types=[
            pltpu.VMEM((CPW, CHUNK), jnp.int32),
        ] + [pltpu.VMEM((CHUNK, HIDDEN), jnp.float32) for _ in range(NBUF)]
          + [pltpu.SemaphoreType.DMA for _ in range(2 * NBUF)],
    )(_emb_body)
    out = run(idx, table)
    return out.reshape(SEQ, BATCH, HIDDEN)


# speedup vs baseline: 1.0552x; 1.0536x over previous
import functools

import jax
import jax.numpy as jnp
from jax import lax
from jax.experimental import pallas as pl
from jax.experimental.pallas import tpu as pltpu
from jax.experimental.pallas import tpu_sc as plsc

SEQ = 200
BATCH = 4096
HIDDEN = 64
WIDE = 2 * HIDDEN
VOCAB = 1000000
TOT = SEQ * BATCH
CHUNK = 128
NC = 2
NS = 16
NW = NC * NS
CPW = SEQ
NBUF = 4
LOOK = 2

RCH = 200
RNCH = VOCAB // RCH
RVIS = 2 * ((RNCH // NW + 2) // 2)


def _repack_body(table_hbm, t2_hbm, r0, r1, sb0, sb1, s0, s1, w0, w1):
    rbufs = (r0, r1)
    sbufs = (sb0, sb1)
    rsems = (s0, s1)
    wsems = (w0, w1)
    wid = lax.axis_index("s") * NC + lax.axis_index("c")

    def rd(k, b):
        cid = wid + k * NW
        return pltpu.make_async_copy(
            table_hbm.at[pl.ds(cid * RCH, RCH)], rbufs[b], rsems[b])

    def wr(k, b):
        cid = wid + k * NW
        return pltpu.make_async_copy(
            sbufs[b], t2_hbm.at[pl.ds(cid * RCH, RCH)], wsems[b])

    def valid(k):
        return wid + k * NW < RNCH

    def spread(b):
        rb, sb = rbufs[b], sbufs[b]

        def rows(r4, carry):
            for rr in range(4):
                r = r4 * 4 + rr
                for j in range(4):
                    sb[r, pl.ds(j * 16, 16)] = rb[r, pl.ds(j * 16, 16)]
            return carry

        lax.fori_loop(0, RCH // 4, rows, 0)

    rd(0, 0).start()

    def group(g, carry):
        for b in range(2):
            k = g * 2 + b
            nb = b ^ 1

            @pl.when(jnp.logical_and(k >= 1, valid(k - 1)))
            def _():
                wr(k - 1, nb).wait()

            @pl.when(valid(k + 1))
            def _():
                rd(k + 1, nb).start()

            @pl.when(valid(k))
            def _():
                rd(k, b).wait()
                spread(b)
                wr(k, b).start()
        return carry

    lax.fori_loop(0, RVIS // 2, group, 0)

    @pl.when(valid(RVIS - 1))
    def _():
        wr(RVIS - 1, (RVIS - 1) % 2).wait()


def _gather_body(idx_hbm, t2_hbm, out_hbm, idx_v, *rest):
    gbufs = rest[:NBUF]
    cbufs = rest[NBUF:NBUF + 2]
    sems = rest[NBUF + 2:2 * NBUF + 2]
    stsems = rest[2 * NBUF + 2:]
    wid = lax.axis_index("s") * NC + lax.axis_index("c")
    col0 = wid * CHUNK

    def out_at(c):
        return out_hbm.at[pl.ds(c * BATCH + col0, CHUNK)]

    def gather(c, b):
        pltpu.make_async_copy(t2_hbm.at[idx_v.at[c]], gbufs[b], sems[b]).start()

    def store(c, cb):
        return pltpu.make_async_copy(cbufs[cb], out_at(c), stsems[cb])

    def compact(b, cb):
        gb, cb = gbufs[b], cbufs[cb]

        def rows(r4, carry):
            for rr in range(4):
                r = r4 * 4 + rr
                for j in range(4):
                    cb[r, pl.ds(j * 16, 16)] = gb[r, pl.ds(j * 16, 16)]
            return carry

        lax.fori_loop(0, CHUNK // 4, rows, 0)

    pltpu.sync_copy(idx_hbm.at[:, pl.ds(col0, CHUNK)], idx_v)

    for c in range(LOOK):
        gather(c, c % NBUF)

    def group(g, carry):
        for b in range(NBUF):
            c = g * NBUF + b
            pb = (b + LOOK) % NBUF
            cb = b % 2

            @pl.when(c + LOOK < CPW)
            def _():
                gather(c + LOOK, pb)

            pltpu.make_async_copy(t2_hbm.at[idx_v.at[c]], gbufs[b], sems[b]).wait()

            @pl.when(c >= 2)
            def _():
                store(c - 2, cb).wait()

            compact(b, cb)
            store(c, cb).start()
        return carry

    lax.fori_loop(0, CPW // NBUF, group, 0)

    for c in range(CPW - 2, CPW):
        store(c, c % 2).wait()


def kernel(input_ids, table):
    mesh = plsc.VectorSubcoreMesh(core_axis_name="c", subcore_axis_name="s")
    repack = functools.partial(
        pl.kernel,
        mesh=mesh,
        out_type=jax.ShapeDtypeStruct((VOCAB, WIDE), jnp.float32),
        scratch_types=[pltpu.VMEM((RCH, HIDDEN), jnp.float32) for _ in range(2)]
        + [pltpu.VMEM((RCH, WIDE), jnp.float32) for _ in range(2)]
        + [pltpu.SemaphoreType.DMA for _ in range(4)],
    )(_repack_body)
    t2 = repack(table)

    gather = functools.partial(
        pl.kernel,
        mesh=mesh,
        out_type=jax.ShapeDtypeStruct((TOT, HIDDEN), jnp.float32),
        scratch_types=[pltpu.VMEM((CPW, CHUNK), jnp.int32)]
        + [pltpu.VMEM((CHUNK, WIDE), jnp.float32) for _ in range(NBUF)]
        + [pltpu.VMEM((CHUNK, HIDDEN), jnp.float32) for _ in range(2)]
        + [pltpu.SemaphoreType.DMA for _ in range(NBUF + 2)],
    )(_gather_body)
    out = gather(input_ids.astype(jnp.int32), t2)
    return out.reshape(SEQ, BATCH, HIDDEN)
